# baseline (speedup 1.0000x reference)
import functools

import jax
import jax.numpy as jnp
from jax import lax
from jax.experimental import pallas as pl
from jax.experimental.pallas import tpu as pltpu
from jax.experimental.pallas import tpu_sc as plsc

B = 16384
D = 16
NU = 1000000
TR = NU * D // 128
NC = 2
NS = 16
NW = NC * NS
BPW = B // NW
CHK = 128
NCK = BPW // CHK
NBLK = CHK // 16
NT = NU // 128
TPS = NT // NW


@functools.cache
def _detile():
    mesh = plsc.VectorSubcoreMesh(core_axis_name="c", subcore_axis_name="s")

    @functools.partial(
        pl.kernel,
        mesh=mesh,
        out_type=tuple(
            jax.ShapeDtypeStruct((TR, 128), jnp.float32) for _ in range(4)
        ),
        scratch_types=(
            [pltpu.VMEM((D, 128), jnp.float32) for _ in range(8)]
            + [pltpu.VMEM((D, 128), jnp.float32) for _ in range(8)]
            + [pltpu.SemaphoreType.DMA] * 4
        ),
        compiler_params=pltpu.CompilerParams(needs_layout_passes=False),
    )
    def detile(t0, t1, t2, t3, x0, x1, x2, x3, o0, o1, o2, o3,
               i00, i01, i10, i11, i20, i21, i30, i31,
               s00, s01, s10, s11, s20, s21, s30, s31,
               rs0, rs1, ws0, ws1):
        tabs = (t0, t1, t2, t3)
        tails = (x0, x1, x2, x3)
        outs = (o0, o1, o2, o3)
        inb = ((i00, i01), (i10, i11), (i20, i21), (i30, i31))
        osb = ((s00, s01), (s10, s11), (s20, s21), (s30, s31))
        rsems = (rs0, rs1)
        wsems = (ws0, ws1)
        wid = lax.axis_index("s") * NC + lax.axis_index("c")
        lane = lax.iota(jnp.int32, 16)

        def fire_reads(tile, par):
            c0 = tile * 128
            for ti in range(4):
                for dd in range(2):
                    pltpu.async_copy(
                        tabs[ti].at[pl.ds(dd * 8, 8), pl.ds(c0, 128)],
                        inb[ti][par].at[pl.ds(dd * 8, 8), :], rsems[par])

        def drain_reads(par):
            for ti in range(4):
                pltpu.make_async_copy(
                    t0.at[:, pl.ds(0, 128)], inb[ti][par], rsems[par]).wait()

        def permute(par, nrr):
            for ti in range(4):
                for rr in range(nrr):
                    for p in range(8):
                        col = jnp.full((16,), 8 * rr + p, jnp.int32)
                        v = plsc.load_gather(inb[ti][par], [lane, col])
                        osb[ti][par][rr, pl.ds(16 * p, 16)] = v

        def fire_writes(tile, par):
            for ti in range(4):
                pltpu.async_copy(
                    osb[ti][par], outs[ti].at[pl.ds(tile * 16, 16), :],
                    wsems[par])

        def drain_writes(par):
            for ti in range(4):
                pltpu.make_async_copy(
                    t0.at[:, pl.ds(0, 128)], osb[ti][par], wsems[par]).wait()

        fire_reads(wid, 0)

        def body(m, _):
            for par in (0, 1):
                k = m * 2 + par
                tile = k * NW + wid
                drain_reads(par)

                @pl.when(k + 1 < TPS)
                def _(par=par, k=k):
                    fire_reads((k + 1) * NW + wid, 1 - par)

                @pl.when(k >= 2)
                def _(par=par):
                    drain_writes(par)

                permute(par, 16)
                fire_writes(tile, par)
            return 0

        lax.fori_loop(0, TPS // 2, body, 0)
        drain_writes(0)
        drain_writes(1)

        @pl.when(wid < 4)
        def _():
            tile = TPS * NW + wid
            fire_reads(tile, 0)
            drain_reads(0)
            permute(0, 16)
            fire_writes(tile, 0)
            drain_writes(0)

        @pl.when(wid == 4)
        def _():
            for ti in range(4):
                pltpu.async_copy(
                    tails[ti], inb[ti][1].at[pl.ds(0, 8), :], rsems[1])
            for ti in range(4):
                pltpu.make_async_copy(
                    t0.at[pl.ds(0, 8), pl.ds(0, 128)],
                    inb[ti][1].at[pl.ds(0, 8), :], rsems[1]).wait()
            for ti in range(4):
                pltpu.async_copy(
                    inb[ti][1].at[pl.ds(0, 8), :],
                    outs[ti].at[pl.ds(NT * 16, 8), :], wsems[1])
            for ti in range(4):
                pltpu.make_async_copy(
                    t0.at[pl.ds(0, 8), pl.ds(0, 128)],
                    inb[ti][1].at[pl.ds(0, 8), :], wsems[1]).wait()

    return detile


@functools.cache
def _nmf_sc():
    mesh = plsc.VectorSubcoreMesh(core_axis_name="c", subcore_axis_name="s")

    @functools.partial(
        pl.kernel,
        mesh=mesh,
        out_type=(jax.ShapeDtypeStruct((B,), jnp.float32),
                  jax.ShapeDtypeStruct((NW, 16), jnp.float32)),
        scratch_types=[
            pltpu.VMEM((BPW,), jnp.int32),
            pltpu.VMEM((BPW,), jnp.int32),
            pltpu.VMEM((BPW,), jnp.int32),
            pltpu.VMEM((BPW,), jnp.int32),
            pltpu.VMEM((CHK, 128), jnp.float32),
            pltpu.VMEM((CHK, 128), jnp.float32),
            pltpu.VMEM((CHK, 128), jnp.float32),
            pltpu.VMEM((CHK, 128), jnp.float32),
            pltpu.VMEM((BPW,), jnp.float32),
            pltpu.VMEM((BPW,), jnp.float32),
            pltpu.VMEM((16,), jnp.float32),
            pltpu.VMEM((2 * D * D * D,), jnp.float32),
            pltpu.VMEM((D * D,), jnp.float32),
            pltpu.SemaphoreType.DMA,
        ],
        compiler_params=pltpu.CompilerParams(needs_layout_passes=False),
    )
    def nmf_sc(user_hbm, item_hbm, rat_hbm, umlp_hbm, imlp_hbm,
               umf_hbm, imf_hbm, w1_hbm, w2_hbm,
               tgt_hbm, lp_hbm,
               uidx, iidx, umaj, imaj, g0, g1, g2, g3,
               rbuf, tbuf, lbuf, wbv, w2v, sem):
        wid = lax.axis_index("s") * NC + lax.axis_index("c")
        pltpu.sync_copy(user_hbm.at[wid], uidx)
        pltpu.sync_copy(item_hbm.at[wid], iidx)
        pltpu.sync_copy(rat_hbm.at[wid], rbuf)
        pltpu.sync_copy(w1_hbm, wbv)
        pltpu.sync_copy(w2_hbm, w2v)

        lane = lax.iota(jnp.int32, 16)

        def majs(g, _):
            s = pl.ds(g * 16, 16)
            umaj[s] = uidx[s] >> 3
            imaj[s] = iidx[s] >> 3
            return 0

        lax.fori_loop(0, BPW // 16, majs, 0)

        def tanh_e(e):
            return (e - 1.0) / (e + 1.0)

        def rsqrt_nr(x):
            i = plsc.bitcast(x, jnp.int32)
            y = plsc.bitcast(0x5F3759DF - (i >> 1), jnp.float32)
            for _ in range(3):
                y = y * (1.5 - 0.5 * x * y * y)
            return y

        lacc = jnp.zeros((16,), jnp.float32)
        for c in range(NCK):
            cs = pl.ds(c * CHK, CHK)
            copies = [
                pltpu.async_copy(t.at[m.at[cs]], g, sem)
                for t, m, g in ((umlp_hbm, umaj, g0), (imlp_hbm, imaj, g1),
                                (umf_hbm, umaj, g2), (imf_hbm, imaj, g3))]
            for cp in copies:
                cp.wait()

            def blk(b, acc, c=c):
                e0 = c * CHK + b * 16
                iu = uidx[pl.ds(e0, 16)]
                ii = iidx[pl.ds(e0, 16)]
                ru = (iu & 7) * 16
                ri = (ii & 7) * 16
                row = b * 16 + lane
                zero = jnp.zeros((16,), jnp.float32)
                h = [zero] * D
                su = sv = suu = svv = suv = zero
                for d in range(D):
                    cu = plsc.load_gather(g0, [row, ru + d])
                    ci = plsc.load_gather(g1, [row, ri + d])
                    xu = plsc.load_gather(g2, [row, ru + d])
                    xv = plsc.load_gather(g3, [row, ri + d])
                    for j in range(D):
                        k = (d * D + j) * D
                        h[j] = (h[j] + wbv[pl.ds(k, 16)] * cu
                                + wbv[pl.ds(D * D * D + k, 16)] * ci)
                    su = su + xu
                    sv = sv + xv
                    suu = suu + xu * xu
                    svv = svv + xv * xv
                    suv = suv + xu * xv
                m = jnp.zeros((16,), jnp.float32)
                for j in range(D):
                    m = m + w2v[pl.ds(j * D, 16)] * tanh_e(jnp.exp(h[j]))
                mlp = tanh_e(jnp.exp(m))
                ssu = suu - su * su * (1.0 / 16.0)
                ssv = svv - sv * sv * (1.0 / 16.0)
                duv = suv - su * sv * (1.0 / 16.0)
                nu = jnp.maximum(ssu * rsqrt_nr(ssu), 1e-12)
                nv = jnp.maximum(ssv * rsqrt_nr(ssv), 1e-12)
                mf = duv / (nu * nv)
                nmf = 0.5 * (mlp + mf)
                rn = rbuf[pl.ds(e0, 16)] * 0.5 - 1.5
                diff = nmf - rn
                tbuf[pl.ds(e0, 16)] = nmf * 2.0 + 3.0
                return acc + diff * diff

            lacc = lax.fori_loop(0, NBLK, blk, lacc)

        lbuf[...] = lacc * (1.0 / B)
        base = wid * BPW
        pltpu.sync_copy(tbuf, tgt_hbm.at[pl.ds(base, BPW)])
        pltpu.sync_copy(lbuf, lp_hbm.at[wid])

    return nmf_sc


def kernel(user, item, rating, U_mlp, I_mlp, U_mf, I_mf, W1, W2):
    user_r = user.astype(jnp.int32).reshape(NW, BPW)
    item_r = item.astype(jnp.int32).reshape(NW, BPW)
    rat_r = rating.reshape(NW, BPW)
    wu = (2.0 * W1[:, :D]).T.reshape(D * D, 1)
    wi = (2.0 * W1[:, D:]).T.reshape(D * D, 1)
    wb = jnp.broadcast_to(jnp.concatenate([wu, wi], axis=0),
                          (2 * D * D, D)).reshape(2 * D * D * D)
    w2b = jnp.broadcast_to((2.0 * W2).reshape(D, 1), (D, D)).reshape(D * D)
    tails = [t[NT * 128:, :].reshape(8, 128)
             for t in (U_mlp, I_mlp, U_mf, I_mf)]
    r0, r1, r2, r3 = _detile()(U_mlp.T, I_mlp.T, U_mf.T, I_mf.T, *tails)
    tgt, lparts = _nmf_sc()(
        user_r, item_r, rat_r, r0, r1, r2, r3, wb, w2b)
    return jnp.sum(lparts), tgt


# speedup vs baseline: 1.7278x; 1.7278x over previous
import functools

import jax
import jax.numpy as jnp
from jax import lax
from jax.experimental import pallas as pl
from jax.experimental.pallas import tpu as pltpu
from jax.experimental.pallas import tpu_sc as plsc

B = 16384
D = 16
NU = 1000000
TR = NU * D // 128
NC = 2
NS = 16
NW = NC * NS
BPW = B // NW
CHK = 128
NCK = BPW // CHK
NBLK = CHK // 16
NT = NU // 128
TPS = NT // NW


@functools.cache
def _detile():
    mesh = plsc.VectorSubcoreMesh(core_axis_name="c", subcore_axis_name="s")

    @functools.partial(
        pl.kernel,
        mesh=mesh,
        out_type=tuple(
            jax.ShapeDtypeStruct((TR, 128), jnp.float32) for _ in range(4)
        ),
        scratch_types=(
            [pltpu.VMEM((D, 128), jnp.float32) for _ in range(8)]
            + [pltpu.VMEM((D, 128), jnp.float32) for _ in range(8)]
            + [pltpu.SemaphoreType.DMA] * 4
        ),
        compiler_params=pltpu.CompilerParams(needs_layout_passes=False),
    )
    def detile(t0, t1, t2, t3, x0, x1, x2, x3, o0, o1, o2, o3,
               i00, i01, i10, i11, i20, i21, i30, i31,
               s00, s01, s10, s11, s20, s21, s30, s31,
               rs0, rs1, ws0, ws1):
        tabs = (t0, t1, t2, t3)
        tails = (x0, x1, x2, x3)
        outs = (o0, o1, o2, o3)
        inb = ((i00, i01), (i10, i11), (i20, i21), (i30, i31))
        osb = ((s00, s01), (s10, s11), (s20, s21), (s30, s31))
        rsems = (rs0, rs1)
        wsems = (ws0, ws1)
        wid = lax.axis_index("s") * NC + lax.axis_index("c")
        lane = lax.iota(jnp.int32, 16)

        def fire_reads(tile, par):
            c0 = tile * 128
            for ti in range(4):
                for dd in range(2):
                    pltpu.async_copy(
                        tabs[ti].at[pl.ds(dd * 8, 8), pl.ds(c0, 128)],
                        inb[ti][par].at[pl.ds(dd * 8, 8), :], rsems[par])

        def drain_reads(par):
            for ti in range(4):
                pltpu.make_async_copy(
                    t0.at[:, pl.ds(0, 128)], inb[ti][par], rsems[par]).wait()

        def permute(par, nrr):
            for rr in range(nrr):
                vs = [
                    plsc.load_gather(
                        inb[ti][par],
                        [lane, jnp.full((16,), 8 * rr + p, jnp.int32)])
                    for ti in range(4) for p in range(8)]
                for ti in range(4):
                    for p in range(8):
                        osb[ti][par][rr, pl.ds(16 * p, 16)] = vs[ti * 8 + p]

        def fire_writes(tile, par):
            for ti in range(4):
                pltpu.async_copy(
                    osb[ti][par], outs[ti].at[pl.ds(tile * 16, 16), :],
                    wsems[par])

        def drain_writes(par):
            for ti in range(4):
                pltpu.make_async_copy(
                    t0.at[:, pl.ds(0, 128)], osb[ti][par], wsems[par]).wait()

        fire_reads(wid, 0)

        def body(m, _):
            for par in (0, 1):
                k = m * 2 + par
                tile = k * NW + wid
                drain_reads(par)

                @pl.when(k + 1 < TPS)
                def _(par=par, k=k):
                    fire_reads((k + 1) * NW + wid, 1 - par)

                @pl.when(k >= 2)
                def _(par=par):
                    drain_writes(par)

                permute(par, 16)
                fire_writes(tile, par)
            return 0

        lax.fori_loop(0, TPS // 2, body, 0)
        drain_writes(0)
        drain_writes(1)

        @pl.when(wid < 4)
        def _():
            tile = TPS * NW + wid
            fire_reads(tile, 0)
            drain_reads(0)
            permute(0, 16)
            fire_writes(tile, 0)
            drain_writes(0)

        @pl.when(wid == 4)
        def _():
            for ti in range(4):
                pltpu.async_copy(
                    tails[ti], inb[ti][1].at[pl.ds(0, 8), :], rsems[1])
            for ti in range(4):
                pltpu.make_async_copy(
                    t0.at[pl.ds(0, 8), pl.ds(0, 128)],
                    inb[ti][1].at[pl.ds(0, 8), :], rsems[1]).wait()
            for ti in range(4):
                pltpu.async_copy(
                    inb[ti][1].at[pl.ds(0, 8), :],
                    outs[ti].at[pl.ds(NT * 16, 8), :], wsems[1])
            for ti in range(4):
                pltpu.make_async_copy(
                    t0.at[pl.ds(0, 8), pl.ds(0, 128)],
                    inb[ti][1].at[pl.ds(0, 8), :], wsems[1]).wait()

    return detile


@functools.cache
def _nmf_sc():
    mesh = plsc.VectorSubcoreMesh(core_axis_name="c", subcore_axis_name="s")

    @functools.partial(
        pl.kernel,
        mesh=mesh,
        out_type=(jax.ShapeDtypeStruct((B,), jnp.float32),
                  jax.ShapeDtypeStruct((NW, 16), jnp.float32)),
        scratch_types=[
            pltpu.VMEM((BPW,), jnp.int32),
            pltpu.VMEM((BPW,), jnp.int32),
            pltpu.VMEM((BPW,), jnp.int32),
            pltpu.VMEM((BPW,), jnp.int32),
            pltpu.VMEM((CHK, 128), jnp.float32),
            pltpu.VMEM((CHK, 128), jnp.float32),
            pltpu.VMEM((CHK, 128), jnp.float32),
            pltpu.VMEM((CHK, 128), jnp.float32),
            pltpu.VMEM((BPW,), jnp.float32),
            pltpu.VMEM((BPW,), jnp.float32),
            pltpu.VMEM((16,), jnp.float32),
            pltpu.VMEM((2 * D * D * D,), jnp.float32),
            pltpu.VMEM((D * D,), jnp.float32),
            pltpu.SemaphoreType.DMA,
        ],
        compiler_params=pltpu.CompilerParams(needs_layout_passes=False),
    )
    def nmf_sc(user_hbm, item_hbm, rat_hbm, umlp_hbm, imlp_hbm,
               umf_hbm, imf_hbm, w1_hbm, w2_hbm,
               tgt_hbm, lp_hbm,
               uidx, iidx, umaj, imaj, g0, g1, g2, g3,
               rbuf, tbuf, lbuf, wbv, w2v, sem):
        wid = lax.axis_index("s") * NC + lax.axis_index("c")
        pltpu.sync_copy(user_hbm.at[wid], uidx)
        pltpu.sync_copy(item_hbm.at[wid], iidx)
        pltpu.sync_copy(rat_hbm.at[wid], rbuf)
        pltpu.sync_copy(w1_hbm, wbv)
        pltpu.sync_copy(w2_hbm, w2v)

        lane = lax.iota(jnp.int32, 16)

        def majs(g, _):
            s = pl.ds(g * 16, 16)
            umaj[s] = uidx[s] >> 3
            imaj[s] = iidx[s] >> 3
            return 0

        lax.fori_loop(0, BPW // 16, majs, 0)

        def tanh_e(e):
            return (e - 1.0) / (e + 1.0)

        def rsqrt_nr(x):
            i = plsc.bitcast(x, jnp.int32)
            y = plsc.bitcast(0x5F3759DF - (i >> 1), jnp.float32)
            for _ in range(3):
                y = y * (1.5 - 0.5 * x * y * y)
            return y

        lacc = jnp.zeros((16,), jnp.float32)
        for c in range(NCK):
            cs = pl.ds(c * CHK, CHK)
            copies = [
                pltpu.async_copy(t.at[m.at[cs]], g, sem)
                for t, m, g in ((umlp_hbm, umaj, g0), (imlp_hbm, imaj, g1),
                                (umf_hbm, umaj, g2), (imf_hbm, imaj, g3))]
            for cp in copies:
                cp.wait()

            def blk(b, acc, c=c):
                e0 = c * CHK + b * 16
                iu = uidx[pl.ds(e0, 16)]
                ii = iidx[pl.ds(e0, 16)]
                ru = (iu & 7) * 16
                ri = (ii & 7) * 16
                row = b * 16 + lane
                zero = jnp.zeros((16,), jnp.float32)
                h = [zero] * D
                su = sv = suu = svv = suv = zero
                for d in range(D):
                    cu = plsc.load_gather(g0, [row, ru + d])
                    ci = plsc.load_gather(g1, [row, ri + d])
                    xu = plsc.load_gather(g2, [row, ru + d])
                    xv = plsc.load_gather(g3, [row, ri + d])
                    for j in range(D):
                        k = (d * D + j) * D
                        h[j] = (h[j] + wbv[pl.ds(k, 16)] * cu
                                + wbv[pl.ds(D * D * D + k, 16)] * ci)
                    su = su + xu
                    sv = sv + xv
                    suu = suu + xu * xu
                    svv = svv + xv * xv
                    suv = suv + xu * xv
                m = jnp.zeros((16,), jnp.float32)
                for j in range(D):
                    m = m + w2v[pl.ds(j * D, 16)] * tanh_e(jnp.exp(h[j]))
                mlp = tanh_e(jnp.exp(m))
                ssu = suu - su * su * (1.0 / 16.0)
                ssv = svv - sv * sv * (1.0 / 16.0)
                duv = suv - su * sv * (1.0 / 16.0)
                nu = jnp.maximum(ssu * rsqrt_nr(ssu), 1e-12)
                nv = jnp.maximum(ssv * rsqrt_nr(ssv), 1e-12)
                mf = duv / (nu * nv)
                nmf = 0.5 * (mlp + mf)
                rn = rbuf[pl.ds(e0, 16)] * 0.5 - 1.5
                diff = nmf - rn
                tbuf[pl.ds(e0, 16)] = nmf * 2.0 + 3.0
                return acc + diff * diff

            lacc = lax.fori_loop(0, NBLK, blk, lacc)

        lbuf[...] = lacc * (1.0 / B)
        base = wid * BPW
        pltpu.sync_copy(tbuf, tgt_hbm.at[pl.ds(base, BPW)])
        pltpu.sync_copy(lbuf, lp_hbm.at[wid])

    return nmf_sc


def kernel(user, item, rating, U_mlp, I_mlp, U_mf, I_mf, W1, W2):
    user_r = user.astype(jnp.int32).reshape(NW, BPW)
    item_r = item.astype(jnp.int32).reshape(NW, BPW)
    rat_r = rating.reshape(NW, BPW)
    wu = (2.0 * W1[:, :D]).T.reshape(D * D, 1)
    wi = (2.0 * W1[:, D:]).T.reshape(D * D, 1)
    wb = jnp.broadcast_to(jnp.concatenate([wu, wi], axis=0),
                          (2 * D * D, D)).reshape(2 * D * D * D)
    w2b = jnp.broadcast_to((2.0 * W2).reshape(D, 1), (D, D)).reshape(D * D)
    tails = [t[NT * 128:, :].reshape(8, 128)
             for t in (U_mlp, I_mlp, U_mf, I_mf)]
    r0, r1, r2, r3 = _detile()(U_mlp.T, I_mlp.T, U_mf.T, I_mf.T, *tails)
    tgt, lparts = _nmf_sc()(
        user_r, item_r, rat_r, r0, r1, r2, r3, wb, w2b)
    return jnp.sum(lparts), tgt
